# baseline (speedup 1.0000x reference)
import functools

import jax
import jax.numpy as jnp
from jax import lax
from jax.experimental import pallas as pl
from jax.experimental.pallas import tpu as pltpu, tpu_sc as plsc

VOCAB = 1000000
DIM = 64
BATCH = 4096
SEQ = 50

_INFO = plsc.get_sparse_core_info()
NC, NS = _INFO.num_cores, _INFO.num_subcores
NW = NC * NS
TOTAL = BATCH * SEQ
PER_W = TOTAL // NW
CHUNK = 128
NCHUNK = PER_W // CHUNK
NBUF = 10

PAIR_ROWS = 500224
LIN_ROWS = 2 * PAIR_ROWS


def _detile_table(word_emb):
    wt = word_emb.T

    def body(xa_ref, xb_ref, o_ref):
        z = jnp.concatenate([xa_ref[...], xb_ref[...]], axis=0)
        o_ref[...] = z.T

    out = pl.pallas_call(
        body,
        grid=(PAIR_ROWS // 512,),
        in_specs=[
            pl.BlockSpec((64, 512), lambda j: (0, 2 * j)),
            pl.BlockSpec((64, 512), lambda j: (0, 2 * j + 1)),
        ],
        out_specs=pl.BlockSpec((512, 128), lambda j: (j, 0)),
        out_shape=jax.ShapeDtypeStruct((PAIR_ROWS, 128), jnp.float32),
    )(wt, wt)
    return out.reshape(LIN_ROWS, DIM)


def _make_gather():
    mesh = plsc.VectorSubcoreMesh(core_axis_name="c", subcore_axis_name="s")

    @functools.partial(
        pl.kernel,
        mesh=mesh,
        out_type=[
            jax.ShapeDtypeStruct((TOTAL, DIM), jnp.float32),
            jax.ShapeDtypeStruct((TOTAL, DIM), jnp.float32),
        ],
        scratch_types=[
            pltpu.VMEM((NCHUNK, CHUNK), jnp.int32),
            pltpu.VMEM((NBUF, CHUNK, DIM), jnp.float32),
            pltpu.SemaphoreType.DMA,
        ],
        compiler_params=pltpu.CompilerParams(use_tc_tiling_on_sc=False),
    )
    def k(table_hbm, pidx_hbm, hidx_hbm, p_out, h_out, idx_v, rows_v, sem):
        wid = lax.axis_index("s") * NC + lax.axis_index("c")
        base = wid * PER_W
        for idx_hbm, out_hbm in ((pidx_hbm, p_out), (hidx_hbm, h_out)):
            pltpu.sync_copy(idx_hbm.at[wid], idx_v)

            def xform(c, carry):
                for kk in range(CHUNK // 16):
                    v = idx_v[c, pl.ds(kk * 16, 16)]
                    lrow = (
                        (v & -1024)
                        + ((v & 511) << 1)
                        + ((v >> 9) & 1)
                    )
                    idx_v[c, pl.ds(kk * 16, 16)] = lrow
                return carry

            lax.fori_loop(0, NCHUNK, xform, 0)

            def prime(b, carry):
                pltpu.async_copy(table_hbm.at[idx_v.at[b]], rows_v.at[b], sem)
                return carry

            lax.fori_loop(0, NBUF, prime, 0)

            def chunk_body(j, carry, out_hbm=out_hbm):
                b = lax.rem(j, NBUF)
                pltpu.make_async_copy(
                    table_hbm.at[idx_v.at[0]], rows_v.at[0], sem
                ).wait()
                pltpu.sync_copy(
                    rows_v.at[b], out_hbm.at[pl.ds(base + j * CHUNK, CHUNK)]
                )

                @pl.when(j + NBUF < NCHUNK)
                def _():
                    pltpu.async_copy(
                        table_hbm.at[idx_v.at[j + NBUF]], rows_v.at[b], sem
                    )

                return carry

            lax.fori_loop(0, NCHUNK, chunk_body, 0)

    return k


_gather = _make_gather()


def kernel(premise, hypothesis, word_emb):
    table_lin = _detile_table(word_emb)
    pidx = premise.T.reshape(NW, NCHUNK, CHUNK)
    hidx = hypothesis.T.reshape(NW, NCHUNK, CHUNK)
    p_rows, h_rows = _gather(table_lin, pidx, hidx)
    p = p_rows.reshape(SEQ, BATCH, DIM).transpose((1, 0, 2))
    h = h_rows.reshape(SEQ, BATCH, DIM).transpose((1, 0, 2))
    return (p, h)


# speedup vs baseline: 1.1260x; 1.1260x over previous
import functools

import jax
import jax.numpy as jnp
from jax import lax
from jax.experimental import pallas as pl
from jax.experimental.pallas import tpu as pltpu, tpu_sc as plsc

VOCAB = 1000000
DIM = 64
BATCH = 4096
SEQ = 50

_INFO = plsc.get_sparse_core_info()
NC, NS = _INFO.num_cores, _INFO.num_subcores
NW = NC * NS
TOTAL = BATCH * SEQ
PER_W = TOTAL // NW
CHUNK = 128
NCHUNK = PER_W // CHUNK
NBUF = 10


def _make_gather():
    mesh = plsc.VectorSubcoreMesh(core_axis_name="c", subcore_axis_name="s")

    @functools.partial(
        pl.kernel,
        mesh=mesh,
        out_type=[
            jax.ShapeDtypeStruct((TOTAL, DIM), jnp.float32),
            jax.ShapeDtypeStruct((TOTAL, DIM), jnp.float32),
        ],
        scratch_types=[
            pltpu.VMEM((NCHUNK, CHUNK), jnp.int32),
            pltpu.VMEM((NBUF, CHUNK, DIM), jnp.float32),
            pltpu.SemaphoreType.DMA,
        ],
        compiler_params=pltpu.CompilerParams(use_tc_tiling_on_sc=False),
    )
    def k(table_hbm, pidx_hbm, hidx_hbm, p_out, h_out, idx_v, rows_v, sem):
        wid = lax.axis_index("s") * NC + lax.axis_index("c")
        base = wid * PER_W
        for idx_hbm, out_hbm in ((pidx_hbm, p_out), (hidx_hbm, h_out)):
            pltpu.sync_copy(idx_hbm.at[wid], idx_v)

            def prime(b, carry):
                pltpu.async_copy(table_hbm.at[idx_v.at[b]], rows_v.at[b], sem)
                return carry

            lax.fori_loop(0, NBUF, prime, 0)

            def chunk_body(j, carry, out_hbm=out_hbm):
                b = lax.rem(j, NBUF)
                pltpu.make_async_copy(
                    table_hbm.at[idx_v.at[0]], rows_v.at[0], sem
                ).wait()
                pltpu.sync_copy(
                    rows_v.at[b], out_hbm.at[pl.ds(base + j * CHUNK, CHUNK)]
                )

                @pl.when(j + NBUF < NCHUNK)
                def _():
                    pltpu.async_copy(
                        table_hbm.at[idx_v.at[j + NBUF]], rows_v.at[b], sem
                    )

                return carry

            lax.fori_loop(0, NCHUNK, chunk_body, 0)

    return k


_gather = _make_gather()


def kernel(premise, hypothesis, word_emb):
    table_lin = jnp.pad(word_emb, ((0, 0), (0, DIM))).reshape(2 * VOCAB, DIM)
    pidx = premise.T.reshape(NW, NCHUNK, CHUNK) * 2
    hidx = hypothesis.T.reshape(NW, NCHUNK, CHUNK) * 2
    p_rows, h_rows = _gather(table_lin, pidx, hidx)
    p = p_rows.reshape(SEQ, BATCH, DIM).transpose((1, 0, 2))
    h = h_rows.reshape(SEQ, BATCH, DIM).transpose((1, 0, 2))
    return (p, h)
